# Initial kernel scaffold; baseline (speedup 1.0000x reference)
import jax
import jax.numpy as jnp
from jax.experimental import pallas as pl


def kernel(pred, target, acc_sum):
    raise NotImplementedError("write your pallas kernel here")


# speedup vs baseline: 3.7023x; 3.7023x over previous
import functools
import jax
import jax.numpy as jnp
import numpy as np
from jax.experimental import pallas as pl
from jax.experimental.pallas import tpu as pltpu

BINS = 30
MOMENTUM = 0.75
LOSS_WEIGHT = 1.0
LANES = 128


def _ghm_kernel(pred_ref, target_ref, accsum_ref, out_ref,
                acc_c, acc_s, *, nblocks, nelem, edges):
    pid = pl.program_id(0)

    @pl.when(pid == 0)
    def _init():
        acc_c[...] = jnp.zeros_like(acc_c)
        acc_s[...] = jnp.zeros_like(acc_s)

    p = pred_ref[...]
    t = target_ref[...]
    bce = jnp.logaddexp(0.0, p) - p * t
    g = jnp.abs(jax.nn.sigmoid(p) - t)
    rows = p.shape[0]
    rid = jax.lax.broadcasted_iota(jnp.int32, p.shape, 0)
    lid = jax.lax.broadcasted_iota(jnp.int32, p.shape, 1)
    eid = (pid * rows + rid) * LANES + lid
    g = jnp.where(eid < nelem, g, -1.0)

    c_parts = []
    s_parts = []
    for b in range(BINS):
        m = g >= edges[b]
        c_parts.append(jnp.sum(jnp.where(m, 1.0, 0.0), axis=0, keepdims=True))
        s_parts.append(jnp.sum(jnp.where(m, bce, 0.0), axis=0, keepdims=True))
    zeros2 = jnp.zeros((2, LANES), dtype=jnp.float32)
    acc_c[...] += jnp.concatenate(c_parts + [zeros2], axis=0)
    acc_s[...] += jnp.concatenate(s_parts + [zeros2], axis=0)

    @pl.when(pid == nblocks - 1)
    def _finalize():
        T_c = jnp.sum(acc_c[...], axis=1, keepdims=True)
        T_s = jnp.sum(acc_s[...], axis=1, keepdims=True)
        zero1 = jnp.zeros((1, 1), dtype=jnp.float32)
        cnt = T_c - jnp.concatenate([T_c[1:], zero1], axis=0)
        sbce = T_s - jnp.concatenate([T_s[1:], zero1], axis=0)
        a = accsum_ref[...][:, 0:1]
        total = jnp.float32(nelem)
        nonempty = cnt > 0
        acc_new = jnp.where(nonempty,
                            MOMENTUM * a + (1.0 - MOMENTUM) * cnt, a)
        safe = jnp.where(nonempty, acc_new, 1.0)
        w = jnp.where(nonempty, total / safe, 0.0)
        n = jnp.sum(jnp.where(nonempty, 1.0, 0.0))
        wsum = jnp.sum(w * sbce)
        denom = jnp.where(n > 0, jnp.maximum(n, 1.0), 1.0)
        out_ref[0, 0] = (wsum / denom) / total * LOSS_WEIGHT


def _ghm_loss(pred, target, acc_sum):
    nelem = pred.size
    nrows = -(-nelem // LANES)
    blk = 512
    while blk > 8 and blk * 2 > nrows:
        blk //= 2
    nrows_pad = -(-nrows // blk) * blk
    nblocks = nrows_pad // blk
    flat = jnp.zeros((nrows_pad * LANES,), jnp.float32)
    p2 = flat.at[:nelem].set(pred.ravel()).reshape(nrows_pad, LANES)
    t2 = flat.at[:nelem].set(target.ravel()).reshape(nrows_pad, LANES)
    a_pad = jnp.zeros((32, LANES), jnp.float32).at[:BINS, 0].set(acc_sum)
    edges = [float(np.float32(b) / np.float32(BINS)) for b in range(BINS)]

    out = pl.pallas_call(
        functools.partial(_ghm_kernel, nblocks=nblocks, nelem=nelem,
                          edges=edges),
        grid=(nblocks,),
        in_specs=[
            pl.BlockSpec((blk, LANES), lambda i: (i, 0)),
            pl.BlockSpec((blk, LANES), lambda i: (i, 0)),
            pl.BlockSpec((32, LANES), lambda i: (0, 0)),
        ],
        out_specs=pl.BlockSpec(memory_space=pltpu.SMEM),
        out_shape=jax.ShapeDtypeStruct((1, 1), jnp.float32),
        scratch_shapes=[
            pltpu.VMEM((32, LANES), jnp.float32),
            pltpu.VMEM((32, LANES), jnp.float32),
        ],
        compiler_params=pltpu.CompilerParams(
            dimension_semantics=("arbitrary",)),
    )(p2, t2, a_pad)
    return out[0, 0]


def kernel(pred, target, acc_sum):
    return _ghm_loss(pred, target, acc_sum)


# speedup vs baseline: 4.0842x; 1.1032x over previous
import functools
import jax
import jax.numpy as jnp
import numpy as np
from jax.experimental import pallas as pl
from jax.experimental.pallas import tpu as pltpu

BINS = 30
MOMENTUM = 0.75
LOSS_WEIGHT = 1.0
LANES = 128


def _ghm_kernel(pred_ref, target_ref, accsum_ref, out_ref,
                acc_c, acc_s, *, nblocks, nelem, edges, mask_rows):
    pid = pl.program_id(0)

    @pl.when(pid == 0)
    def _init():
        acc_c[...] = jnp.zeros_like(acc_c)
        acc_s[...] = jnp.zeros_like(acc_s)

    p = pred_ref[...]
    t = target_ref[...]
    lanes = p.shape[1]
    bce = jnp.logaddexp(0.0, p) - p * t
    g = jnp.abs(jax.nn.sigmoid(p) - t)
    if mask_rows is not None:
        rid = pid * p.shape[0] + jax.lax.broadcasted_iota(
            jnp.int32, p.shape, 0)
        g = jnp.where(rid < mask_rows, g, -1.0)

    c_parts = []
    s_parts = []
    for b in range(BINS):
        mf = jnp.where(g >= edges[b], 1.0, 0.0)
        c_parts.append(jnp.sum(mf, axis=0, keepdims=True))
        s_parts.append(jnp.sum(mf * bce, axis=0, keepdims=True))
    zeros2 = jnp.zeros((2, lanes), dtype=jnp.float32)
    acc_c[...] += jnp.concatenate(c_parts + [zeros2], axis=0)
    acc_s[...] += jnp.concatenate(s_parts + [zeros2], axis=0)

    @pl.when(pid == nblocks - 1)
    def _finalize():
        T_c = jnp.sum(acc_c[...], axis=1, keepdims=True)
        T_s = jnp.sum(acc_s[...], axis=1, keepdims=True)
        zero1 = jnp.zeros((1, 1), dtype=jnp.float32)
        cnt = T_c - jnp.concatenate([T_c[1:], zero1], axis=0)
        sbce = T_s - jnp.concatenate([T_s[1:], zero1], axis=0)
        a = accsum_ref[...][:, 0:1]
        total = jnp.float32(nelem)
        nonempty = cnt > 0
        acc_new = jnp.where(nonempty,
                            MOMENTUM * a + (1.0 - MOMENTUM) * cnt, a)
        safe = jnp.where(nonempty, acc_new, 1.0)
        w = jnp.where(nonempty, total / safe, 0.0)
        n = jnp.sum(jnp.where(nonempty, 1.0, 0.0))
        wsum = jnp.sum(w * sbce)
        denom = jnp.where(n > 0, jnp.maximum(n, 1.0), 1.0)
        out_ref[0, 0] = (wsum / denom) / total * LOSS_WEIGHT


def _pick_block(nrows):
    for b in range(min(nrows, 2048), 7, -1):
        if b % 8 == 0 and nrows % b == 0:
            return b
    return 0


def _ghm_loss(pred, target, acc_sum):
    nelem = pred.size
    cols = pred.shape[-1]
    p2 = pred.reshape(-1, cols)
    t2 = target.reshape(-1, cols)
    nrows = p2.shape[0]
    blk = _pick_block(nrows)
    mask_rows = None
    if blk == 0:
        blk = 512 if nrows >= 512 else 8
        mask_rows = nrows
    nrows_pad = -(-nrows // blk) * blk
    npad = nrows_pad - nrows
    if npad:
        p2 = jnp.pad(p2, ((0, npad), (0, 0)))
        t2 = jnp.pad(t2, ((0, npad), (0, 0)))
    nblocks = nrows_pad // blk
    a_pad = jnp.zeros((32, cols), jnp.float32).at[:BINS, 0].set(acc_sum)
    edges = [float(np.float32(b) / np.float32(BINS)) for b in range(BINS)]

    out = pl.pallas_call(
        functools.partial(_ghm_kernel, nblocks=nblocks, nelem=nelem,
                          edges=edges, mask_rows=mask_rows),
        grid=(nblocks,),
        in_specs=[
            pl.BlockSpec((blk, cols), lambda i: (i, 0)),
            pl.BlockSpec((blk, cols), lambda i: (i, 0)),
            pl.BlockSpec((32, cols), lambda i: (0, 0)),
        ],
        out_specs=pl.BlockSpec(memory_space=pltpu.SMEM),
        out_shape=jax.ShapeDtypeStruct((1, 1), jnp.float32),
        scratch_shapes=[
            pltpu.VMEM((32, cols), jnp.float32),
            pltpu.VMEM((32, cols), jnp.float32),
        ],
        compiler_params=pltpu.CompilerParams(
            dimension_semantics=("arbitrary",)),
    )(p2, t2, a_pad)
    return out[0, 0]


def kernel(pred, target, acc_sum):
    return _ghm_loss(pred, target, acc_sum)
